# baseline (speedup 1.0000x reference)
import functools

import jax
import jax.numpy as jnp
import numpy as np
from jax import lax
from jax.experimental import pallas as pl
from jax.experimental.pallas import tpu as pltpu
from jax.experimental.pallas import tpu_sc as plsc

N_BINS = 15
_BOUNDS = np.linspace(0.0, 1.0, N_BINS + 1, dtype=np.float32)
_CB = 40
_SUB = 8

_NSUB = 16
_LANE = 16
_NPAD = 50176
_CHUNK = _NPAD // _NSUB


def _argmax_kernel(x_ref, lab_ref, out_ref, m_ref, b_ref):
    i = pl.program_id(0)
    nb = pl.num_programs(0)

    @pl.when(i == 0)
    def _init():
        m_ref[...] = jnp.full_like(m_ref, -jnp.inf)
        b_ref[...] = jnp.zeros_like(b_ref)

    m = m_ref[...]
    b = b_ref[...]
    for j in range(_CB // _SUB):
        sub = x_ref[_SUB * j:_SUB * (j + 1), :]
        upd = sub > m
        m = jnp.where(upd, sub, m)
        b = jnp.where(upd, i * _CB + j * _SUB, b)
    m_ref[...] = m
    b_ref[...] = b

    @pl.when(i == nb - 1)
    def _finish():
        mm = m_ref[...]
        idx = b_ref[...] + jax.lax.broadcasted_iota(jnp.int32, mm.shape, 0)
        gmax = jnp.max(mm, axis=0, keepdims=True)
        ji = jnp.where(mm == gmax, idx, jnp.int32(1 << 30))
        fmi = jnp.min(ji, axis=0, keepdims=True)
        out_ref[...] = (fmi == lab_ref[...]).astype(jnp.float32)


def _tc_accuracies(xt, lab2):
    c, n = xt.shape
    return pl.pallas_call(
        _argmax_kernel,
        grid=(c // _CB,),
        in_specs=[
            pl.BlockSpec((_CB, n), lambda i: (i, 0)),
            pl.BlockSpec((1, n), lambda i: (0, 0)),
        ],
        out_specs=pl.BlockSpec((1, n), lambda i: (0, 0)),
        out_shape=jax.ShapeDtypeStruct((1, n), jnp.float32),
        scratch_shapes=[
            pltpu.VMEM((_SUB, n), jnp.float32),
            pltpu.VMEM((_SUB, n), jnp.int32),
        ],
    )(xt, lab2)


def _sc_hist(conf_hbm, acc_hbm, out_hbm,
             conf_v, acc_v, cnt_v, sumc_v, suma_v, part_v, shared, tmp_v,
             ece_v):
    sid = lax.axis_index("s")
    base = sid * _CHUNK
    pltpu.sync_copy(conf_hbm.at[pl.ds(base, _CHUNK)], conf_v)
    pltpu.sync_copy(acc_hbm.at[pl.ds(base, _CHUNK)], acc_v)
    zeros = jnp.zeros((_LANE,), jnp.float32)
    cnt_v[...] = zeros
    sumc_v[...] = zeros
    suma_v[...] = zeros
    ones = jnp.ones((_LANE,), jnp.float32)

    def body(k, carry):
        c = conf_v[pl.ds(k * _LANE, _LANE)]
        a = acc_v[pl.ds(k * _LANE, _LANE)]
        pos = jnp.zeros((_LANE,), jnp.int32)
        for kk in range(N_BINS):
            pos = pos + jnp.where(c > float(_BOUNDS[kk]), 1, 0)
        idx = jnp.where(pos == 0, N_BINS, pos - 1)
        plsc.addupdate_scatter(cnt_v, [idx], ones)
        plsc.addupdate_scatter(sumc_v, [idx], c)
        plsc.addupdate_scatter(suma_v, [idx], a)
        return carry

    lax.fori_loop(0, _CHUNK // _LANE, body, 0)

    part_v[pl.ds(0, _LANE)] = cnt_v[...]
    part_v[pl.ds(_LANE, _LANE)] = sumc_v[...]
    part_v[pl.ds(2 * _LANE, _LANE)] = suma_v[...]
    pltpu.sync_copy(part_v, shared.at[pl.ds(sid * 3 * _LANE, 3 * _LANE)])
    plsc.subcore_barrier()

    @pl.when(sid == 0)
    def _reduce():
        pltpu.sync_copy(shared, tmp_v)
        cnt = jnp.zeros((_LANE,), jnp.float32)
        sumc = jnp.zeros((_LANE,), jnp.float32)
        suma = jnp.zeros((_LANE,), jnp.float32)
        for w in range(_NSUB):
            cnt = cnt + tmp_v[pl.ds(w * 3 * _LANE, _LANE)]
            sumc = sumc + tmp_v[pl.ds(w * 3 * _LANE + _LANE, _LANE)]
            suma = suma + tmp_v[pl.ds(w * 3 * _LANE + 2 * _LANE, _LANE)]
        safe = jnp.where(cnt > 0, cnt, 1.0)
        lane = lax.broadcasted_iota(jnp.int32, (_LANE,), 0)
        ok = (cnt > 0) & (lane < N_BINS)
        contrib = jnp.where(
            ok,
            jnp.abs(sumc / safe - suma / safe) * (cnt / 50000.0),
            0.0,
        )
        ece_v[...] = jnp.full((_LANE,), jnp.sum(contrib))
        pltpu.sync_copy(ece_v, out_hbm)


_SC_MESH = plsc.VectorSubcoreMesh(
    core_axis_name="c", subcore_axis_name="s", num_cores=1)

_sc_ece = functools.partial(
    pl.kernel,
    mesh=_SC_MESH,
    compiler_params=pltpu.CompilerParams(needs_layout_passes=False),
    out_type=jax.ShapeDtypeStruct((_LANE,), jnp.float32),
    scratch_types=[
        pltpu.VMEM((_CHUNK,), jnp.float32),
        pltpu.VMEM((_CHUNK,), jnp.float32),
        pltpu.VMEM((_LANE,), jnp.float32),
        pltpu.VMEM((_LANE,), jnp.float32),
        pltpu.VMEM((_LANE,), jnp.float32),
        pltpu.VMEM((3 * _LANE,), jnp.float32),
        pltpu.VMEM_SHARED((_NSUB * 3 * _LANE,), jnp.float32),
        pltpu.VMEM((_NSUB * 3 * _LANE,), jnp.float32),
        pltpu.VMEM((_LANE,), jnp.float32),
    ],
)(_sc_hist)


def kernel(softmaxes, confidences, labels):
    n, c = softmaxes.shape
    xt = softmaxes.T
    lab2 = labels.astype(jnp.int32).reshape(1, n)
    acc = _tc_accuracies(xt, lab2)
    conf_p = jnp.pad(confidences, (0, _NPAD - n))
    acc_p = jnp.pad(acc.reshape(n), (0, _NPAD - n))
    out = _sc_ece(conf_p, acc_p)
    return out[0:1]


# speedup vs baseline: 1.0423x; 1.0423x over previous
import functools

import jax
import jax.numpy as jnp
import numpy as np
from jax import lax
from jax.experimental import pallas as pl
from jax.experimental.pallas import tpu as pltpu
from jax.experimental.pallas import tpu_sc as plsc

N_BINS = 15
_BOUNDS = np.linspace(0.0, 1.0, N_BINS + 1, dtype=np.float32)
_CB = 40
_SUB = 8

_NSUB = 16
_LANE = 16
_NPAD = 50176
_CHUNK = _NPAD // _NSUB


def _argmax_kernel(x_ref, lab_ref, out_ref, m_ref, b_ref):
    i = pl.program_id(0)
    nb = pl.num_programs(0)

    @pl.when(i == 0)
    def _init():
        m_ref[...] = jnp.full_like(m_ref, -jnp.inf)
        b_ref[...] = jnp.zeros_like(b_ref)

    m = m_ref[...]
    b = b_ref[...]
    for j in range(_CB // _SUB):
        sub = x_ref[_SUB * j:_SUB * (j + 1), :]
        upd = sub > m
        m = jnp.where(upd, sub, m)
        b = jnp.where(upd, i * _CB + j * _SUB, b)
    m_ref[...] = m
    b_ref[...] = b

    @pl.when(i == nb - 1)
    def _finish():
        mm = m_ref[...]
        idx = b_ref[...] + jax.lax.broadcasted_iota(jnp.int32, mm.shape, 0)
        gmax = jnp.max(mm, axis=0, keepdims=True)
        ji = jnp.where(mm == gmax, idx, jnp.int32(1 << 30))
        fmi = jnp.min(ji, axis=0, keepdims=True)
        out_ref[...] = (fmi == lab_ref[...]).astype(jnp.float32)


def _tc_accuracies(xt, lab2):
    c, n = xt.shape
    return pl.pallas_call(
        _argmax_kernel,
        grid=(c // _CB,),
        in_specs=[
            pl.BlockSpec((_CB, n), lambda i: (i, 0)),
            pl.BlockSpec((1, n), lambda i: (0, 0)),
        ],
        out_specs=pl.BlockSpec((1, n), lambda i: (0, 0)),
        out_shape=jax.ShapeDtypeStruct((1, n), jnp.float32),
        scratch_shapes=[
            pltpu.VMEM((_SUB, n), jnp.float32),
            pltpu.VMEM((_SUB, n), jnp.int32),
        ],
    )(xt, lab2)


def _sc_hist(conf_hbm, out_hbm,
             conf_v, cnt_v, sumc_v, part_v, shared, tmp_v, stage_v):
    sid = lax.axis_index("s")
    base = sid * _CHUNK
    pltpu.sync_copy(conf_hbm.at[pl.ds(base, _CHUNK)], conf_v)
    zeros = jnp.zeros((_LANE,), jnp.float32)
    cnt_v[...] = zeros
    sumc_v[...] = zeros
    ones = jnp.ones((_LANE,), jnp.float32)

    def body(k, carry):
        c = conf_v[pl.ds(k * _LANE, _LANE)]
        pos = jnp.zeros((_LANE,), jnp.int32)
        for kk in range(N_BINS):
            pos = pos + jnp.where(c > float(_BOUNDS[kk]), 1, 0)
        idx = jnp.where(pos == 0, N_BINS, pos - 1)
        plsc.addupdate_scatter(cnt_v, [idx], ones)
        plsc.addupdate_scatter(sumc_v, [idx], c)
        return carry

    lax.fori_loop(0, _CHUNK // _LANE, body, 0)

    part_v[pl.ds(0, _LANE)] = cnt_v[...]
    part_v[pl.ds(_LANE, _LANE)] = sumc_v[...]
    pltpu.sync_copy(part_v, shared.at[pl.ds(sid * 2 * _LANE, 2 * _LANE)])
    plsc.subcore_barrier()

    @pl.when(sid == 0)
    def _reduce():
        pltpu.sync_copy(shared, tmp_v)
        cnt = jnp.zeros((_LANE,), jnp.float32)
        sumc = jnp.zeros((_LANE,), jnp.float32)
        for w in range(_NSUB):
            cnt = cnt + tmp_v[pl.ds(w * 2 * _LANE, _LANE)]
            sumc = sumc + tmp_v[pl.ds(w * 2 * _LANE + _LANE, _LANE)]
        lane = lax.broadcasted_iota(jnp.int32, (_LANE,), 0)
        plsc.store_scatter(stage_v, [2 * lane], cnt)
        plsc.store_scatter(stage_v, [2 * lane + 1], sumc)
        pltpu.sync_copy(stage_v, out_hbm)


_SC_MESH = plsc.VectorSubcoreMesh(
    core_axis_name="c", subcore_axis_name="s", num_cores=1)

_sc_conf_hist = functools.partial(
    pl.kernel,
    mesh=_SC_MESH,
    compiler_params=pltpu.CompilerParams(needs_layout_passes=False),
    out_type=jax.ShapeDtypeStruct((2 * _LANE,), jnp.float32),
    scratch_types=[
        pltpu.VMEM((_CHUNK,), jnp.float32),
        pltpu.VMEM((_LANE,), jnp.float32),
        pltpu.VMEM((_LANE,), jnp.float32),
        pltpu.VMEM((2 * _LANE,), jnp.float32),
        pltpu.VMEM_SHARED((_NSUB * 2 * _LANE,), jnp.float32),
        pltpu.VMEM((_NSUB * 2 * _LANE,), jnp.float32),
        pltpu.VMEM((2 * _LANE,), jnp.float32),
    ],
)(_sc_hist)


def _combine_kernel(conf_ref, acc_ref, part_ref, bounds_ref, out_ref):
    acc = acc_ref[...]
    conf = conf_ref[...]
    n = conf.shape[1]
    conf_b = jnp.broadcast_to(conf, (_SUB, n))
    acc_b = jnp.broadcast_to(acc, (_SUB, n))
    ece = jnp.zeros((1, 1), jnp.float32)
    for g in range(2):
        lob = bounds_ref[_SUB * g:_SUB * (g + 1), 0:1]
        hib = bounds_ref[_SUB * g:_SUB * (g + 1), 1:2]
        mask = ((conf_b > lob) & (conf_b <= hib)).astype(jnp.float32)
        suma = jnp.sum(mask * acc_b, axis=1, keepdims=True)
        cnt = part_ref[_SUB * g:_SUB * (g + 1), 0:1]
        sumc = part_ref[_SUB * g:_SUB * (g + 1), 1:2]
        safe = jnp.where(cnt > 0, cnt, 1.0)
        contrib = jnp.where(
            cnt > 0,
            jnp.abs(sumc / safe - suma / safe) * (cnt / n),
            0.0,
        )
        ece += jnp.sum(contrib).reshape(1, 1)
    out_ref[...] = ece


def _tc_combine(conf2, acc, part2, bounds):
    n = conf2.shape[1]
    return pl.pallas_call(
        _combine_kernel,
        grid=(1,),
        in_specs=[
            pl.BlockSpec((1, n), lambda i: (0, 0)),
            pl.BlockSpec((1, n), lambda i: (0, 0)),
            pl.BlockSpec((16, 2), lambda i: (0, 0)),
            pl.BlockSpec((16, 2), lambda i: (0, 0)),
        ],
        out_specs=pl.BlockSpec((1, 1), lambda i: (0, 0)),
        out_shape=jax.ShapeDtypeStruct((1, 1), jnp.float32),
    )(conf2, acc, part2, bounds)


def kernel(softmaxes, confidences, labels):
    n, c = softmaxes.shape
    xt = softmaxes.T
    lab2 = labels.astype(jnp.int32).reshape(1, n)
    conf2 = confidences.reshape(1, n)
    conf_p = jnp.pad(confidences, (0, _NPAD - n))
    bnp = np.full((16, 2), 2.0, dtype=np.float32)
    bnp[:N_BINS, 0] = _BOUNDS[:-1]
    bnp[:N_BINS, 1] = _BOUNDS[1:]
    bounds = jnp.asarray(bnp)

    part = _sc_conf_hist(conf_p)
    acc = _tc_accuracies(xt, lab2)
    part2 = part.reshape(16, 2)
    out = _tc_combine(conf2, acc, part2, bounds)
    return out.reshape(1)
